# baseline (speedup 1.0000x reference)
import functools

import jax
import jax.numpy as jnp
from jax import lax
from jax.experimental import pallas as pl
from jax.experimental.pallas import tpu as pltpu
from jax.experimental.pallas import tpu_sc as plsc

N = 10000
E = 320000
R = 3
H = 128
HH = H // 2

NC = 2
NS = 16
NR = N * R

CA = 64
GCA = 5024
EPAD = GCA * CA
KA = GCA // NS
GJ = KA // 2
CD = 128
GCD = E // CD
KD = -(-GCD // (NC * NS))
NRP = 30336
NP = NRP // R
ROWS_T = NRP // NS

_mesh = plsc.VectorSubcoreMesh(core_axis_name="c", subcore_axis_name="s")
_sc_params = pltpu.CompilerParams(use_tc_tiling_on_sc=False)


def _deg_body(seg2_hbm, ones_hbm, z_hbm, out_hbm,
              accum, didx0, didx1, ones_v, si0, si1):
    c = lax.axis_index("c")
    s = lax.axis_index("s")
    w = c * NS + s
    didx = (didx0, didx1)
    si = (si0, si1)
    pltpu.sync_copy(z_hbm, accum.at[pl.ds(s * ROWS_T, ROWS_T)])
    pltpu.sync_copy(ones_hbm, ones_v)

    def g_of(k):
        return k * (NC * NS) + w

    def issue_idx(k, b):
        @pl.when(g_of(k) < GCD)
        def _():
            pltpu.async_copy(seg2_hbm.at[pl.ds(g_of(k) * CD, CD)],
                             didx[b], si[b])

    issue_idx(0, 0)
    issue_idx(1, 1)
    plsc.subcore_barrier()

    @pl.loop(0, KD)
    def _(k):
        @pl.when(g_of(k) < GCD)
        def _():
            def scat(bb):
                pltpu.make_async_copy(
                    seg2_hbm.at[pl.ds(g_of(k) * CD, CD)], didx[bb],
                    si[bb]).wait()
                pltpu.sync_copy(ones_v, accum.at[didx[bb]], add=True)

            @pl.when(lax.rem(k, 2) == 0)
            def _():
                scat(0)
                issue_idx(k + 2, 0)

            @pl.when(lax.rem(k, 2) == 1)
            def _():
                scat(1)
                issue_idx(k + 2, 1)

    plsc.subcore_barrier()
    sl = pl.ds(s * ROWS_T, ROWS_T)
    pltpu.sync_copy(accum.at[sl], out_hbm.at[c].at[sl])


def _deg_pass(seg2, ones_hbm, z_hbm):
    k = pl.kernel(
        _deg_body,
        out_type=jax.ShapeDtypeStruct((NC, NRP, 16), jnp.float32),
        mesh=_mesh,
        compiler_params=_sc_params,
        scratch_types=[
            pltpu.VMEM_SHARED((NRP, 16), jnp.float32),
            pltpu.VMEM((CD,), jnp.int32),
            pltpu.VMEM((CD,), jnp.int32),
            pltpu.VMEM((CD, 16), jnp.float32),
            pltpu.SemaphoreType.DMA,
            pltpu.SemaphoreType.DMA,
        ],
    )
    return k(seg2, ones_hbm, z_hbm)


def _agg_body(hs2_hbm, comb_hbm, z_hbm, out_hbm, accum,
              rows0, rows1, st0, st1, st2,
              sg0, sg1, ss0, ss1, sp0, sp1, sp2, sz):
    c = lax.axis_index("c")
    s = lax.axis_index("s")
    rows = (rows0, rows1)
    stg = (st0, st1, st2)
    sg = (sg0, sg1)
    ss = (ss0, ss1)
    sp = (sp0, sp1, sp2)
    tab = hs2_hbm.at[pl.ds(c, 2 * N - 1)]

    def stg_ref(j):
        return comb_hbm.at[pl.ds(s * KA + 2 * j, 2)]

    def issue_stg(j, u):
        pltpu.async_copy(stg_ref(j), stg[u], sp[u])

    def wait_stg(j, u):
        pltpu.make_async_copy(stg_ref(j), stg[u], sp[u]).wait()

    def issue_gather(S, m, b):
        pltpu.async_copy(tab.at[S.at[m].at[0]], rows[b], sg[b])

    def wait_gather(S, m, b):
        pltpu.make_async_copy(tab.at[S.at[m].at[0]], rows[b], sg[b]).wait()

    def issue_scatter(S, m, b):
        pltpu.async_copy(rows[b], accum.at[S.at[m].at[1]], ss[b], add=True)

    def wait_scatter(S, m, b):
        pltpu.make_async_copy(rows[b], accum.at[S.at[m].at[1]], ss[b]).wait()

    zcopy = pltpu.make_async_copy(z_hbm, accum.at[pl.ds(s * ROWS_T, ROWS_T)],
                                  sz)
    zcopy.start()
    issue_stg(0, 0)
    issue_stg(1, 1)
    wait_stg(0, 0)
    issue_gather(st0, 0, 0)
    zcopy.wait()
    plsc.subcore_barrier()

    @pl.loop(0, GJ)
    def _(j):
        def group(S, Sn, So, u, un, uo):
            wait_gather(S, 0, 0)
            issue_scatter(S, 0, 0)

            @pl.when(j >= 1)
            def _():
                wait_scatter(So, 1, 1)

            @pl.when(j + 2 < GJ)
            def _():
                issue_stg(j + 2, uo)

            issue_gather(S, 1, 1)
            wait_gather(S, 1, 1)
            issue_scatter(S, 1, 1)
            wait_scatter(S, 0, 0)

            @pl.when(j + 1 < GJ)
            def _():
                wait_stg(j + 1, un)
                issue_gather(Sn, 0, 0)

        for u in range(3):
            @pl.when(lax.rem(j, 3) == u)
            def _(u=u):
                group(stg[u], stg[(u + 1) % 3], stg[(u + 2) % 3],
                      u, (u + 1) % 3, (u + 2) % 3)

    u_last = (GJ - 1) % 3
    wait_scatter(stg[u_last], 1, 1)
    plsc.subcore_barrier()
    sl = pl.ds(s * ROWS_T, ROWS_T)
    pltpu.sync_copy(accum.at[sl], out_hbm.at[c].at[sl])


def _agg_pass(hs2, comb, z_hbm):
    k = pl.kernel(
        _agg_body,
        out_type=jax.ShapeDtypeStruct((NC, NRP, HH), jnp.float32),
        mesh=_mesh,
        compiler_params=_sc_params,
        scratch_types=[
            pltpu.VMEM_SHARED((NRP, HH), jnp.float32),
            pltpu.VMEM((CA, HH), jnp.float32),
            pltpu.VMEM((CA, HH), jnp.float32),
            pltpu.VMEM((2, 2, CA), jnp.int32),
            pltpu.VMEM((2, 2, CA), jnp.int32),
            pltpu.VMEM((2, 2, CA), jnp.int32),
            pltpu.SemaphoreType.DMA,
            pltpu.SemaphoreType.DMA,
            pltpu.SemaphoreType.DMA,
            pltpu.SemaphoreType.DMA,
            pltpu.SemaphoreType.DMA,
            pltpu.SemaphoreType.DMA,
            pltpu.SemaphoreType.DMA,
            pltpu.SemaphoreType.DMA,
        ],
    )
    return k(hs2, comb, z_hbm)


BN = 1000


def _inv_from_deg(degT_ref, K_ref):
    d = degT_ref[0] + degT_ref[1]
    cols = jnp.concatenate([d[0][:, 0:1], d[1][:, 0:1], d[2][:, 0:1]], axis=1)
    inv3 = 1.0 / jnp.maximum(cols, 1.0)
    return jnp.dot(inv3, K_ref[...], preferred_element_type=jnp.float32)


def _mm(a_ref, iv, V_ref, b_ref, relu):
    acc = jnp.dot(a_ref[0] * iv, V_ref[0], preferred_element_type=jnp.float32)
    acc = acc + jnp.dot(a_ref[1] * iv, V_ref[1],
                        preferred_element_type=jnp.float32)
    acc = acc + b_ref[...]
    if relu:
        acc = jnp.maximum(acc, 0.0)
    return acc


def _layer0_body(degT_ref, a_ref, K_ref, V_ref, b_ref, h_ref, inv_ref):
    iv = _inv_from_deg(degT_ref, K_ref)
    inv_ref[...] = iv
    h_ref[...] = _mm(a_ref, iv, V_ref, b_ref, True)


def _layer0_pass(degT, aggT, K, V, b):
    return pl.pallas_call(
        _layer0_body,
        grid=(N // BN,),
        in_specs=[
            pl.BlockSpec((NC, R, BN, 16), lambda i: (0, 0, i, 0)),
            pl.BlockSpec((NC, BN, R * HH), lambda i: (0, i, 0)),
            pl.BlockSpec((R, R * HH), lambda i: (0, 0)),
            pl.BlockSpec((NC, R * HH, H), lambda i: (0, 0, 0)),
            pl.BlockSpec((1, H), lambda i: (0, 0)),
        ],
        out_specs=[
            pl.BlockSpec((BN, H), lambda i: (i, 0)),
            pl.BlockSpec((BN, R * HH), lambda i: (i, 0)),
        ],
        out_shape=[
            jax.ShapeDtypeStruct((N, H), jnp.float32),
            jax.ShapeDtypeStruct((N, R * HH), jnp.float32),
        ],
    )(degT, aggT, K, V, b)


def _layer_body(relu, a_ref, iv_ref, V_ref, b_ref, h_ref):
    h_ref[...] = _mm(a_ref, iv_ref[...], V_ref, b_ref, relu)


def _layer_pass(aggT, inv_exp, V, b, relu):
    return pl.pallas_call(
        functools.partial(_layer_body, relu),
        grid=(N // BN,),
        in_specs=[
            pl.BlockSpec((NC, BN, R * HH), lambda i: (0, i, 0)),
            pl.BlockSpec((BN, R * HH), lambda i: (i, 0)),
            pl.BlockSpec((NC, R * HH, H), lambda i: (0, 0, 0)),
            pl.BlockSpec((1, H), lambda i: (0, 0)),
        ],
        out_specs=pl.BlockSpec((BN, H), lambda i: (i, 0)),
        out_shape=jax.ShapeDtypeStruct((N, H), jnp.float32),
    )(aggT, inv_exp, V, b)


def kernel(x, edge_index, edge_type, W1, W2, b0, b1, b2):
    src = edge_index[0]
    dst = edge_index[1]
    et = edge_type
    seg = dst * R + et
    seg2 = et * NP + dst
    pad = EPAD - E
    srcp = jnp.concatenate([2 * src, jnp.zeros((pad,), jnp.int32)])
    segp = jnp.concatenate([seg, jnp.full((pad,), NR, jnp.int32)])
    comb = jnp.stack([srcp.reshape(GCA, CA),
                      segp.reshape(GCA, CA)], axis=1)

    ones16 = jnp.ones((CD, 16), jnp.float32)
    z16 = jnp.zeros((ROWS_T, 16), jnp.float32)
    z64 = jnp.zeros((ROWS_T, HH), jnp.float32)
    K = jnp.repeat(jnp.eye(R, dtype=jnp.float32), HH, axis=1)

    S0 = jnp.tile(jnp.concatenate(
        [jnp.eye(HH, dtype=jnp.float32),
         jnp.zeros((HH, HH), jnp.float32)], axis=1), (R, 1))
    S1 = jnp.tile(jnp.concatenate(
        [jnp.zeros((HH, HH), jnp.float32),
         jnp.eye(HH, dtype=jnp.float32)], axis=1), (R, 1))
    V0 = jnp.stack([S0, S1])
    V1 = jnp.stack([W1[:, :HH, :].reshape(R * HH, H),
                    W1[:, HH:, :].reshape(R * HH, H)])
    V2 = jnp.stack([W2[:, :HH, :].reshape(R * HH, H),
                    W2[:, HH:, :].reshape(R * HH, H)])

    degT = _deg_pass(seg2, ones16, z16)
    agg0 = _agg_pass(x.reshape(NC * N, HH), comb, z64)
    h1, inv_exp = _layer0_pass(degT.reshape(NC, R, NP, 16),
                               agg0.reshape(NC, NP, R * HH),
                               K, V0, b0.reshape(1, H))
    agg1 = _agg_pass(h1.reshape(NC * N, HH), comb, z64)
    h2 = _layer_pass(agg1.reshape(NC, NP, R * HH), inv_exp,
                     V1, b1.reshape(1, H), True)
    agg2 = _agg_pass(h2.reshape(NC * N, HH), comb, z64)
    return _layer_pass(agg2.reshape(NC, NP, R * HH), inv_exp,
                       V2, b2.reshape(1, H), False)


# speedup vs baseline: 1.0830x; 1.0830x over previous
import functools

import jax
import jax.numpy as jnp
from jax import lax
from jax.experimental import pallas as pl
from jax.experimental.pallas import tpu as pltpu
from jax.experimental.pallas import tpu_sc as plsc

N = 10000
E = 320000
R = 3
H = 128
HH = H // 2

NC = 2
NS = 16
NR = N * R

CA = 64
GCA = 5008
EPAD = GCA * CA
KA = GCA // NS
CD = 128
GCD = E // CD
KD = -(-GCD // (NC * NS))
NRP = 30336
NP = NRP // R
ROWS_T = NRP // NS

_mesh = plsc.VectorSubcoreMesh(core_axis_name="c", subcore_axis_name="s")
_sc_params = pltpu.CompilerParams(use_tc_tiling_on_sc=False)


def _deg_body(seg2_hbm, ones_hbm, z_hbm, out_hbm,
              accum, didx0, didx1, ones_v, si0, si1):
    c = lax.axis_index("c")
    s = lax.axis_index("s")
    w = c * NS + s
    didx = (didx0, didx1)
    si = (si0, si1)
    pltpu.sync_copy(z_hbm, accum.at[pl.ds(s * ROWS_T, ROWS_T)])
    pltpu.sync_copy(ones_hbm, ones_v)

    def g_of(k):
        return k * (NC * NS) + w

    def issue_idx(k, b):
        @pl.when(g_of(k) < GCD)
        def _():
            pltpu.async_copy(seg2_hbm.at[pl.ds(g_of(k) * CD, CD)],
                             didx[b], si[b])

    issue_idx(0, 0)
    issue_idx(1, 1)
    plsc.subcore_barrier()

    @pl.loop(0, KD)
    def _(k):
        @pl.when(g_of(k) < GCD)
        def _():
            def scat(bb):
                pltpu.make_async_copy(
                    seg2_hbm.at[pl.ds(g_of(k) * CD, CD)], didx[bb],
                    si[bb]).wait()
                pltpu.sync_copy(ones_v, accum.at[didx[bb]], add=True)

            @pl.when(lax.rem(k, 2) == 0)
            def _():
                scat(0)
                issue_idx(k + 2, 0)

            @pl.when(lax.rem(k, 2) == 1)
            def _():
                scat(1)
                issue_idx(k + 2, 1)

    plsc.subcore_barrier()
    sl = pl.ds(s * ROWS_T, ROWS_T)
    pltpu.sync_copy(accum.at[sl], out_hbm.at[c].at[sl])


def _deg_pass(seg2, ones_hbm, z_hbm):
    k = pl.kernel(
        _deg_body,
        out_type=jax.ShapeDtypeStruct((NC, NRP, 16), jnp.float32),
        mesh=_mesh,
        compiler_params=_sc_params,
        scratch_types=[
            pltpu.VMEM_SHARED((NRP, 16), jnp.float32),
            pltpu.VMEM((CD,), jnp.int32),
            pltpu.VMEM((CD,), jnp.int32),
            pltpu.VMEM((CD, 16), jnp.float32),
            pltpu.SemaphoreType.DMA,
            pltpu.SemaphoreType.DMA,
        ],
    )
    return k(seg2, ones_hbm, z_hbm)


def _agg_body(hs2_hbm, comb_hbm, z_hbm, out_hbm, accum,
              rows0, rows1, idx0, idx1, idx2,
              sg0, sg1, ss0, ss1, si0, si1, si2):
    c = lax.axis_index("c")
    s = lax.axis_index("s")
    rows = (rows0, rows1)
    idx = (idx0, idx1, idx2)
    sg = (sg0, sg1)
    ss = (ss0, ss1)
    si = (si0, si1, si2)
    tab = hs2_hbm.at[pl.ds(c, 2 * N - 1)]

    def src_of(k):
        return comb_hbm.at[k * NS + s]

    def issue_idx(k, i):
        pltpu.async_copy(src_of(k), idx[i], si[i])

    def wait_idx(k, i):
        pltpu.make_async_copy(src_of(k), idx[i], si[i]).wait()

    def issue_gather(k, i, b):
        pltpu.async_copy(tab.at[idx[i].at[0]], rows[b], sg[b])

    def wait_gather(k, i, b):
        pltpu.make_async_copy(tab.at[idx[i].at[0]], rows[b], sg[b]).wait()

    def issue_scatter(k, i, b):
        pltpu.async_copy(rows[b], accum.at[idx[i].at[1]], ss[b], add=True)

    def wait_scatter(k, i, b):
        pltpu.make_async_copy(rows[b], accum.at[idx[i].at[1]], ss[b]).wait()

    zcopy = pltpu.make_async_copy(z_hbm, accum.at[pl.ds(s * ROWS_T, ROWS_T)],
                                  si[2])
    zcopy.start()
    issue_idx(0, 0)
    issue_idx(1, 1)
    wait_idx(0, 0)
    issue_gather(0, 0, 0)
    zcopy.wait()
    plsc.subcore_barrier()

    @pl.loop(0, KA)
    def _(k):
        def step(i, b):
            ii = (i + 1) % 3
            io = (i + 2) % 3
            q = 1 - b
            wait_gather(k, i, b)
            issue_scatter(k, i, b)

            @pl.when(k + 1 < KA)
            def _():
                wait_idx(k + 1, ii)

            @pl.when(k >= 1)
            def _():
                wait_scatter(k - 1, io, q)

            @pl.when(k + 1 < KA)
            def _():
                issue_gather(k + 1, ii, q)

            @pl.when(k + 2 < KA)
            def _():
                issue_idx(k + 2, io)

        for i in range(3):
            @pl.when(lax.rem(k, 3) == i)
            def _(i=i):
                for b in range(2):
                    @pl.when(lax.rem(k, 2) == b)
                    def _(i=i, b=b):
                        step(i, b)

    wait_scatter(KA - 1, (KA - 1) % 3, (KA - 1) % 2)
    plsc.subcore_barrier()
    sl = pl.ds(s * ROWS_T, ROWS_T)
    pltpu.sync_copy(accum.at[sl], out_hbm.at[c].at[sl])


def _agg_pass(hs2, comb, z_hbm):
    k = pl.kernel(
        _agg_body,
        out_type=jax.ShapeDtypeStruct((NC, NRP, HH), jnp.float32),
        mesh=_mesh,
        compiler_params=_sc_params,
        scratch_types=[
            pltpu.VMEM_SHARED((NRP, HH), jnp.float32),
            pltpu.VMEM((CA, HH), jnp.float32),
            pltpu.VMEM((CA, HH), jnp.float32),
            pltpu.VMEM((2, CA), jnp.int32),
            pltpu.VMEM((2, CA), jnp.int32),
            pltpu.VMEM((2, CA), jnp.int32),
            pltpu.SemaphoreType.DMA,
            pltpu.SemaphoreType.DMA,
            pltpu.SemaphoreType.DMA,
            pltpu.SemaphoreType.DMA,
            pltpu.SemaphoreType.DMA,
            pltpu.SemaphoreType.DMA,
            pltpu.SemaphoreType.DMA,
        ],
    )
    return k(hs2, comb, z_hbm)


BN = 2000


def _inv_from_deg(degT_ref, K_ref):
    d = degT_ref[0] + degT_ref[1]
    cols = jnp.concatenate([d[0][:, 0:1], d[1][:, 0:1], d[2][:, 0:1]], axis=1)
    inv3 = 1.0 / jnp.maximum(cols, 1.0)
    return jnp.dot(inv3, K_ref[...], preferred_element_type=jnp.float32)


def _mm(a_ref, iv, V_ref, b_ref, relu):
    acc = jnp.dot(a_ref[0] * iv, V_ref[0], preferred_element_type=jnp.float32)
    acc = acc + jnp.dot(a_ref[1] * iv, V_ref[1],
                        preferred_element_type=jnp.float32)
    acc = acc + b_ref[...]
    if relu:
        acc = jnp.maximum(acc, 0.0)
    return acc


def _layer0_body(degT_ref, a_ref, K_ref, V_ref, b_ref, h_ref, inv_ref):
    iv = _inv_from_deg(degT_ref, K_ref)
    inv_ref[...] = iv
    h_ref[...] = _mm(a_ref, iv, V_ref, b_ref, True)


def _layer0_pass(degT, aggT, K, V, b):
    return pl.pallas_call(
        _layer0_body,
        grid=(N // BN,),
        in_specs=[
            pl.BlockSpec((NC, R, BN, 16), lambda i: (0, 0, i, 0)),
            pl.BlockSpec((NC, BN, R * HH), lambda i: (0, i, 0)),
            pl.BlockSpec((R, R * HH), lambda i: (0, 0)),
            pl.BlockSpec((NC, R * HH, H), lambda i: (0, 0, 0)),
            pl.BlockSpec((1, H), lambda i: (0, 0)),
        ],
        out_specs=[
            pl.BlockSpec((BN, H), lambda i: (i, 0)),
            pl.BlockSpec((BN, R * HH), lambda i: (i, 0)),
        ],
        out_shape=[
            jax.ShapeDtypeStruct((N, H), jnp.float32),
            jax.ShapeDtypeStruct((N, R * HH), jnp.float32),
        ],
    )(degT, aggT, K, V, b)


def _layer_body(relu, a_ref, iv_ref, V_ref, b_ref, h_ref):
    h_ref[...] = _mm(a_ref, iv_ref[...], V_ref, b_ref, relu)


def _layer_pass(aggT, inv_exp, V, b, relu):
    return pl.pallas_call(
        functools.partial(_layer_body, relu),
        grid=(N // BN,),
        in_specs=[
            pl.BlockSpec((NC, BN, R * HH), lambda i: (0, i, 0)),
            pl.BlockSpec((BN, R * HH), lambda i: (i, 0)),
            pl.BlockSpec((NC, R * HH, H), lambda i: (0, 0, 0)),
            pl.BlockSpec((1, H), lambda i: (0, 0)),
        ],
        out_specs=pl.BlockSpec((BN, H), lambda i: (i, 0)),
        out_shape=jax.ShapeDtypeStruct((N, H), jnp.float32),
    )(aggT, inv_exp, V, b)


def kernel(x, edge_index, edge_type, W1, W2, b0, b1, b2):
    src = edge_index[0]
    dst = edge_index[1]
    et = edge_type
    seg = dst * R + et
    seg2 = et * NP + dst
    pad = EPAD - E
    srcp = jnp.concatenate([2 * src, jnp.zeros((pad,), jnp.int32)])
    segp = jnp.concatenate([seg, jnp.full((pad,), NR, jnp.int32)])
    comb = jnp.stack([srcp.reshape(GCA, CA),
                      segp.reshape(GCA, CA)], axis=1)

    ones16 = jnp.ones((CD, 16), jnp.float32)
    z16 = jnp.zeros((ROWS_T, 16), jnp.float32)
    z64 = jnp.zeros((ROWS_T, HH), jnp.float32)
    K = jnp.repeat(jnp.eye(R, dtype=jnp.float32), HH, axis=1)

    S0 = jnp.tile(jnp.concatenate(
        [jnp.eye(HH, dtype=jnp.float32),
         jnp.zeros((HH, HH), jnp.float32)], axis=1), (R, 1))
    S1 = jnp.tile(jnp.concatenate(
        [jnp.zeros((HH, HH), jnp.float32),
         jnp.eye(HH, dtype=jnp.float32)], axis=1), (R, 1))
    V0 = jnp.stack([S0, S1])
    V1 = jnp.stack([W1[:, :HH, :].reshape(R * HH, H),
                    W1[:, HH:, :].reshape(R * HH, H)])
    V2 = jnp.stack([W2[:, :HH, :].reshape(R * HH, H),
                    W2[:, HH:, :].reshape(R * HH, H)])

    degT = _deg_pass(seg2, ones16, z16)
    agg0 = _agg_pass(x.reshape(NC * N, HH), comb, z64)
    h1, inv_exp = _layer0_pass(degT.reshape(NC, R, NP, 16),
                               agg0.reshape(NC, NP, R * HH),
                               K, V0, b0.reshape(1, H))
    agg1 = _agg_pass(h1.reshape(NC * N, HH), comb, z64)
    h2 = _layer_pass(agg1.reshape(NC, NP, R * HH), inv_exp,
                     V1, b1.reshape(1, H), True)
    agg2 = _agg_pass(h2.reshape(NC * N, HH), comb, z64)
    return _layer_pass(agg2.reshape(NC, NP, R * HH), inv_exp,
                       V2, b2.reshape(1, H), False)
